# baseline (speedup 1.0000x reference)
import functools

import jax
import jax.numpy as jnp
from jax import lax
from jax.experimental import pallas as pl
from jax.experimental.pallas import tpu as pltpu
from jax.experimental.pallas import tpu_sc as plsc

NEG = float(-3.0e38)
Q = 64
D = 768
N = 100000
K = 64
BLK = 4096
NPAD = 102400
CH = 6400
NCH = NPAD // CH
GRP = 8
NGRP = CH // (GRP * 16)
CAP = 256
BUF = 640
NVB = BUF // 16
IMAX = 2147483647


def _pool_body(tok_ref, mask_ref, qe_ref):
    m = mask_ref[...].astype(jnp.float32)
    s = jnp.sum(tok_ref[...] * m[..., None], axis=1)
    qe_ref[...] = s / jnp.sum(m, axis=1)[:, None]


def _scores_body(qe_ref, corpus_ref, out_ref):
    i = pl.program_id(0)
    s = lax.dot_general(qe_ref[...], corpus_ref[...],
                        (((1,), (1,)), ((), ())),
                        preferred_element_type=jnp.float32)
    col = i * BLK + lax.broadcasted_iota(jnp.int32, s.shape, 1)
    out_ref[...] = jnp.where(col < N, s, jnp.float32(NEG))


def _splat_f(x):
    return jnp.broadcast_to(x, (16,)).astype(jnp.float32)


def _splat(x, dtype):
    return jnp.broadcast_to(x, (16,)).astype(dtype)


def _topk_body(scores_hbm, out_val_hbm, out_idx_hbm,
               chunk_a, chunk_b, cval, cidx, csort, tval, tidx, oval, oidx,
               sem_a, sem_b):
    wid = lax.axis_index("s") * 2 + lax.axis_index("c")

    def rebuild(t_off):
        _, off = t_off
        def conv_body(j, _):
            v = cval[pl.ds(j * 16, 16)]
            b = plsc.bitcast(v, jnp.uint32)
            s = jnp.where(b >= jnp.uint32(0x80000000), ~b,
                          b | jnp.uint32(0x80000000))
            csort[pl.ds(j * 16, 16)] = s
            return 0
        lax.fori_loop(0, NVB, conv_body, 0)

        def bs_body(_, lohi):
            lo, hi = lohi
            mid = lo + ((hi - lo + jnp.uint32(1)) >> jnp.uint32(1))
            def cnt_body(j, acc):
                s = csort[pl.ds(j * 16, 16)]
                return acc + (s >= _splat(mid, jnp.uint32)).astype(jnp.int32)
            acc = lax.fori_loop(0, NVB, cnt_body, jnp.zeros((16,), jnp.int32))
            cnt = jnp.sum(acc)
            return lax.cond(cnt >= K,
                            lambda: (mid, hi),
                            lambda: (lo, mid - jnp.uint32(1)))
        lo, _ = lax.fori_loop(0, 32, bs_body,
                              (jnp.uint32(0), jnp.uint32(0xFF800000)))

        lov = _splat(lo, jnp.uint32)
        bits = jnp.where((lov & jnp.uint32(0x80000000)) != jnp.uint32(0),
                         lov & jnp.uint32(0x7FFFFFFF), ~lov)
        new_t = jnp.max(plsc.bitcast(bits, jnp.float32))
        nts = _splat_f(new_t)

        def fill_body(j, _):
            tval[pl.ds(j * 16, 16)] = _splat_f(jnp.float32(NEG))
            return 0
        lax.fori_loop(0, NVB, fill_body, 0)

        def comp_body(j, o):
            v = cval[pl.ds(j * 16, 16)]
            ix = cidx[pl.ds(j * 16, 16)]
            m = v >= nts
            plsc.store_compressed(tval.at[pl.ds(o, 16)], v, mask=m)
            plsc.store_compressed(tidx.at[pl.ds(o, 16)], ix, mask=m)
            return o + jnp.sum(m.astype(jnp.int32))
        new_off = lax.fori_loop(0, NVB, comp_body, jnp.int32(0))

        def copy_body(j, _):
            cval[pl.ds(j * 16, 16)] = tval[pl.ds(j * 16, 16)]
            cidx[pl.ds(j * 16, 16)] = tidx[pl.ds(j * 16, 16)]
            return 0
        lax.fori_loop(0, NVB, copy_body, 0)
        return new_t, new_off

    def query_body(qi, _):
        q = wid * 2 + qi

        def init_body(j, _):
            cval[pl.ds(j * 16, 16)] = _splat_f(jnp.float32(NEG))
            return 0
        lax.fori_loop(0, NVB, init_body, 0)

        def start(buf, sem, c):
            pltpu.async_copy(scores_hbm.at[q, pl.ds(c * CH, CH)], buf, sem)

        def wait(buf, sem, c):
            pltpu.make_async_copy(scores_hbm.at[q, pl.ds(c * CH, CH)],
                                  buf, sem).wait()

        def scan_chunk(chunk, c, t_off):
            def group_body(g, t_off):
                t, off = t_off
                base = g * GRP * 16
                acc = _splat_f(jnp.float32(NEG))
                for j in range(GRP):
                    acc = jnp.maximum(acc, chunk[pl.ds(base + j * 16, 16)])
                gm = jnp.max(acc)

                def slow(off):
                    ts = _splat_f(t)
                    o = off
                    for j in range(GRP):
                        v = chunk[pl.ds(base + j * 16, 16)]
                        m = v >= ts
                        iv = (_splat(c * CH + base + j * 16, jnp.int32)
                              + lax.iota(jnp.int32, 16))
                        plsc.store_compressed(cval.at[pl.ds(o, 16)], v, mask=m)
                        plsc.store_compressed(cidx.at[pl.ds(o, 16)], iv, mask=m)
                        o = o + jnp.sum(m.astype(jnp.int32))
                    return o

                off = lax.cond(gm >= t, slow, lambda o: o, off)
                return lax.cond(off > CAP, rebuild, lambda x: x, (t, off))

            return lax.fori_loop(0, NGRP, group_body, t_off)

        start(chunk_a, sem_a, jnp.int32(0))

        def pair_body(cp, t_off):
            c0 = 2 * cp
            c1 = c0 + 1
            c2 = jnp.minimum(c1 + 1, NCH - 1)
            wait(chunk_a, sem_a, c0)
            start(chunk_b, sem_b, c1)
            t_off = scan_chunk(chunk_a, c0, t_off)
            wait(chunk_b, sem_b, c1)
            start(chunk_a, sem_a, c2)
            t_off = scan_chunk(chunk_b, c1, t_off)
            return t_off

        t, off = lax.fori_loop(0, NCH // 2, pair_body,
                               (jnp.float32(NEG), jnp.int32(0)))
        wait(chunk_a, sem_a, jnp.int32(NCH - 1))

        t, off = rebuild((t, off))
        nv = (off + jnp.int32(15)) >> jnp.int32(4)

        def ext_body(s, _):
            def max_body(j, acc):
                return jnp.maximum(acc, cval[pl.ds(j * 16, 16)])
            m = jnp.max(lax.fori_loop(0, nv, max_body,
                                      _splat_f(jnp.float32(NEG))))
            ms = _splat_f(m)

            def idx_body(j, acc):
                v = cval[pl.ds(j * 16, 16)]
                ix = cidx[pl.ds(j * 16, 16)]
                return jnp.minimum(acc, jnp.where(v == ms, ix,
                                                  _splat(IMAX, jnp.int32)))
            bi = jnp.min(lax.fori_loop(0, nv, idx_body,
                                       _splat(IMAX, jnp.int32)))
            bs = _splat(bi, jnp.int32)

            def clr_body(j, _):
                v = cval[pl.ds(j * 16, 16)]
                ix = cidx[pl.ds(j * 16, 16)]
                cval[pl.ds(j * 16, 16)] = jnp.where(
                    (v == ms) & (ix == bs), _splat_f(jnp.float32(NEG)), v)
                return 0
            lax.fori_loop(0, nv, clr_body, 0)

            lane = lax.iota(jnp.int32, 16) == (s & jnp.int32(15))
            sv = _splat(s, jnp.int32)
            plsc.store_scatter(oval, [sv], ms, mask=lane)
            plsc.store_scatter(oidx, [sv], bs, mask=lane)
            return 0
        lax.fori_loop(0, K, ext_body, 0)

        pltpu.sync_copy(oval, out_val_hbm.at[q])
        pltpu.sync_copy(oidx, out_idx_hbm.at[q])
        return 0

    lax.fori_loop(0, 2, query_body, 0)


def kernel(query_token_emb, query_mask, corpus_embeddings, num_retrieve):
    qe = pl.pallas_call(
        _pool_body,
        out_shape=jax.ShapeDtypeStruct((Q, D), jnp.float32),
    )(query_token_emb, query_mask)

    grid = NPAD // BLK
    scores = pl.pallas_call(
        _scores_body,
        grid=(grid,),
        in_specs=[
            pl.BlockSpec((Q, D), lambda i: (0, 0)),
            pl.BlockSpec((BLK, D), lambda i: (i, 0)),
        ],
        out_specs=pl.BlockSpec((Q, BLK), lambda i: (0, i)),
        out_shape=jax.ShapeDtypeStruct((Q, NPAD), jnp.float32),
    )(qe, corpus_embeddings)

    mesh = plsc.VectorSubcoreMesh(core_axis_name="c", subcore_axis_name="s",
                                  num_cores=2, num_subcores=16)
    topk = functools.partial(
        pl.kernel,
        out_type=(jax.ShapeDtypeStruct((Q, K), jnp.float32),
                  jax.ShapeDtypeStruct((Q, K), jnp.int32)),
        mesh=mesh,
        compiler_params=pltpu.CompilerParams(needs_layout_passes=False),
        scratch_types=[
            pltpu.VMEM((CH,), jnp.float32),
            pltpu.VMEM((CH,), jnp.float32),
            pltpu.VMEM((BUF,), jnp.float32),
            pltpu.VMEM((BUF,), jnp.int32),
            pltpu.VMEM((BUF,), jnp.uint32),
            pltpu.VMEM((BUF,), jnp.float32),
            pltpu.VMEM((BUF,), jnp.int32),
            pltpu.VMEM((K,), jnp.float32),
            pltpu.VMEM((K,), jnp.int32),
            pltpu.SemaphoreType.DMA,
            pltpu.SemaphoreType.DMA,
        ],
    )(_topk_body)
    top_vals, top_idx = topk(scores)
    return top_vals, top_idx


# speedup vs baseline: 1.0108x; 1.0108x over previous
import functools

import jax
import jax.numpy as jnp
from jax import lax
from jax.experimental import pallas as pl
from jax.experimental.pallas import tpu as pltpu
from jax.experimental.pallas import tpu_sc as plsc

NEG = float(-3.0e38)
Q = 64
D = 768
N = 100000
K = 64
BLK = 4096
NPAD = 102400
CH = 6400
NCH = NPAD // CH
GRP = 16
NGRP = CH // (GRP * 16)
CAP = 256
BUF = 640
NVB = BUF // 16
IMAX = 2147483647


def _pool_body(tok_ref, mask_ref, qe_ref):
    m = mask_ref[...].astype(jnp.float32)
    s = jnp.sum(tok_ref[...] * m[..., None], axis=1)
    qe_ref[...] = s / jnp.sum(m, axis=1)[:, None]


def _scores_body(qe_ref, corpus_ref, out_ref):
    i = pl.program_id(0)
    s = lax.dot_general(qe_ref[...], corpus_ref[...],
                        (((1,), (1,)), ((), ())),
                        preferred_element_type=jnp.float32)
    col = i * BLK + lax.broadcasted_iota(jnp.int32, s.shape, 1)
    out_ref[...] = jnp.where(col < N, s, jnp.float32(NEG))


def _splat_f(x):
    return jnp.broadcast_to(x, (16,)).astype(jnp.float32)


def _splat(x, dtype):
    return jnp.broadcast_to(x, (16,)).astype(dtype)


def _topk_body(scores_hbm, out_val_hbm, out_idx_hbm,
               chunk_a, chunk_b, cval, cidx, csort, tval, tidx, oval, oidx,
               sem_a, sem_b):
    wid = lax.axis_index("s") * 2 + lax.axis_index("c")

    def rebuild(t_off):
        _, off = t_off
        def conv_body(j, _):
            v = cval[pl.ds(j * 16, 16)]
            b = plsc.bitcast(v, jnp.uint32)
            s = jnp.where(b >= jnp.uint32(0x80000000), ~b,
                          b | jnp.uint32(0x80000000))
            csort[pl.ds(j * 16, 16)] = s
            return 0
        lax.fori_loop(0, NVB, conv_body, 0)

        def bs_body(_, lohi):
            lo, hi = lohi
            mid = lo + ((hi - lo + jnp.uint32(1)) >> jnp.uint32(1))
            def cnt_body(j, acc):
                s = csort[pl.ds(j * 16, 16)]
                return acc + (s >= _splat(mid, jnp.uint32)).astype(jnp.int32)
            acc = lax.fori_loop(0, NVB, cnt_body, jnp.zeros((16,), jnp.int32))
            cnt = jnp.sum(acc)
            return lax.cond(cnt >= K,
                            lambda: (mid, hi),
                            lambda: (lo, mid - jnp.uint32(1)))
        lo, _ = lax.fori_loop(0, 32, bs_body,
                              (jnp.uint32(0), jnp.uint32(0xFF800000)))

        lov = _splat(lo, jnp.uint32)
        bits = jnp.where((lov & jnp.uint32(0x80000000)) != jnp.uint32(0),
                         lov & jnp.uint32(0x7FFFFFFF), ~lov)
        new_t = jnp.max(plsc.bitcast(bits, jnp.float32))
        nts = _splat_f(new_t)

        def fill_body(j, _):
            tval[pl.ds(j * 16, 16)] = _splat_f(jnp.float32(NEG))
            return 0
        lax.fori_loop(0, NVB, fill_body, 0)

        def comp_body(j, o):
            v = cval[pl.ds(j * 16, 16)]
            ix = cidx[pl.ds(j * 16, 16)]
            m = v >= nts
            plsc.store_compressed(tval.at[pl.ds(o, 16)], v, mask=m)
            plsc.store_compressed(tidx.at[pl.ds(o, 16)], ix, mask=m)
            return o + jnp.sum(m.astype(jnp.int32))
        new_off = lax.fori_loop(0, NVB, comp_body, jnp.int32(0))

        def copy_body(j, _):
            cval[pl.ds(j * 16, 16)] = tval[pl.ds(j * 16, 16)]
            cidx[pl.ds(j * 16, 16)] = tidx[pl.ds(j * 16, 16)]
            return 0
        lax.fori_loop(0, NVB, copy_body, 0)
        return new_t, new_off

    def query_body(qi, _):
        q = wid * 2 + qi

        def init_body(j, _):
            cval[pl.ds(j * 16, 16)] = _splat_f(jnp.float32(NEG))
            return 0
        lax.fori_loop(0, NVB, init_body, 0)

        def start(buf, sem, c):
            pltpu.async_copy(scores_hbm.at[q, pl.ds(c * CH, CH)], buf, sem)

        def wait(buf, sem, c):
            pltpu.make_async_copy(scores_hbm.at[q, pl.ds(c * CH, CH)],
                                  buf, sem).wait()

        def scan_chunk(chunk, c, t_off):
            def group_body(g, t_off):
                t, off = t_off
                base = g * GRP * 16
                acc = _splat_f(jnp.float32(NEG))
                for j in range(GRP):
                    acc = jnp.maximum(acc, chunk[pl.ds(base + j * 16, 16)])
                hit = jnp.any(acc >= _splat_f(t))

                def slow(off):
                    ts = _splat_f(t)
                    o = off
                    for j in range(GRP):
                        v = chunk[pl.ds(base + j * 16, 16)]
                        m = v >= ts
                        iv = (_splat(c * CH + base + j * 16, jnp.int32)
                              + lax.iota(jnp.int32, 16))
                        plsc.store_compressed(cval.at[pl.ds(o, 16)], v, mask=m)
                        plsc.store_compressed(cidx.at[pl.ds(o, 16)], iv, mask=m)
                        o = o + jnp.sum(m.astype(jnp.int32))
                    return o

                off = lax.cond(hit, slow, lambda o: o, off)
                return lax.cond(off > CAP, rebuild, lambda x: x, (t, off))

            return lax.fori_loop(0, NGRP, group_body, t_off)

        start(chunk_a, sem_a, jnp.int32(0))

        def pair_body(cp, t_off):
            c0 = 2 * cp
            c1 = c0 + 1
            c2 = jnp.minimum(c1 + 1, NCH - 1)
            wait(chunk_a, sem_a, c0)
            start(chunk_b, sem_b, c1)
            t_off = scan_chunk(chunk_a, c0, t_off)
            wait(chunk_b, sem_b, c1)
            start(chunk_a, sem_a, c2)
            t_off = scan_chunk(chunk_b, c1, t_off)
            return t_off

        t, off = lax.fori_loop(0, NCH // 2, pair_body,
                               (jnp.float32(NEG), jnp.int32(0)))
        wait(chunk_a, sem_a, jnp.int32(NCH - 1))

        t, off = rebuild((t, off))
        nv = (off + jnp.int32(15)) >> jnp.int32(4)

        def ext_body(s, _):
            def max_body(j, acc):
                return jnp.maximum(acc, cval[pl.ds(j * 16, 16)])
            m = jnp.max(lax.fori_loop(0, nv, max_body,
                                      _splat_f(jnp.float32(NEG))))
            ms = _splat_f(m)

            def idx_body(j, acc):
                v = cval[pl.ds(j * 16, 16)]
                ix = cidx[pl.ds(j * 16, 16)]
                return jnp.minimum(acc, jnp.where(v == ms, ix,
                                                  _splat(IMAX, jnp.int32)))
            bi = jnp.min(lax.fori_loop(0, nv, idx_body,
                                       _splat(IMAX, jnp.int32)))
            bs = _splat(bi, jnp.int32)

            def clr_body(j, _):
                v = cval[pl.ds(j * 16, 16)]
                ix = cidx[pl.ds(j * 16, 16)]
                cval[pl.ds(j * 16, 16)] = jnp.where(
                    (v == ms) & (ix == bs), _splat_f(jnp.float32(NEG)), v)
                return 0
            lax.fori_loop(0, nv, clr_body, 0)

            lane = lax.iota(jnp.int32, 16) == (s & jnp.int32(15))
            sv = _splat(s, jnp.int32)
            plsc.store_scatter(oval, [sv], ms, mask=lane)
            plsc.store_scatter(oidx, [sv], bs, mask=lane)
            return 0
        lax.fori_loop(0, K, ext_body, 0)

        pltpu.sync_copy(oval, out_val_hbm.at[q])
        pltpu.sync_copy(oidx, out_idx_hbm.at[q])
        return 0

    lax.fori_loop(0, 2, query_body, 0)


def kernel(query_token_emb, query_mask, corpus_embeddings, num_retrieve):
    qe = pl.pallas_call(
        _pool_body,
        out_shape=jax.ShapeDtypeStruct((Q, D), jnp.float32),
    )(query_token_emb, query_mask)

    grid = NPAD // BLK
    scores = pl.pallas_call(
        _scores_body,
        grid=(grid,),
        in_specs=[
            pl.BlockSpec((Q, D), lambda i: (0, 0)),
            pl.BlockSpec((BLK, D), lambda i: (i, 0)),
        ],
        out_specs=pl.BlockSpec((Q, BLK), lambda i: (0, i)),
        out_shape=jax.ShapeDtypeStruct((Q, NPAD), jnp.float32),
    )(qe, corpus_embeddings)

    mesh = plsc.VectorSubcoreMesh(core_axis_name="c", subcore_axis_name="s",
                                  num_cores=2, num_subcores=16)
    topk = functools.partial(
        pl.kernel,
        out_type=(jax.ShapeDtypeStruct((Q, K), jnp.float32),
                  jax.ShapeDtypeStruct((Q, K), jnp.int32)),
        mesh=mesh,
        compiler_params=pltpu.CompilerParams(needs_layout_passes=False),
        scratch_types=[
            pltpu.VMEM((CH,), jnp.float32),
            pltpu.VMEM((CH,), jnp.float32),
            pltpu.VMEM((BUF,), jnp.float32),
            pltpu.VMEM((BUF,), jnp.int32),
            pltpu.VMEM((BUF,), jnp.uint32),
            pltpu.VMEM((BUF,), jnp.float32),
            pltpu.VMEM((BUF,), jnp.int32),
            pltpu.VMEM((K,), jnp.float32),
            pltpu.VMEM((K,), jnp.int32),
            pltpu.SemaphoreType.DMA,
            pltpu.SemaphoreType.DMA,
        ],
    )(_topk_body)
    top_vals, top_idx = topk(scores)
    return top_vals, top_idx


# speedup vs baseline: 1.0187x; 1.0079x over previous
import functools

import jax
import jax.numpy as jnp
from jax import lax
from jax.experimental import pallas as pl
from jax.experimental.pallas import tpu as pltpu
from jax.experimental.pallas import tpu_sc as plsc

NEG = float(-3.0e38)
Q = 64
D = 768
N = 100000
K = 64
BLK = 4096
NPAD = 102400
CH = 6400
NCH = NPAD // CH
GRP = 16
NGRP = CH // (GRP * 16)
CAP = 256
BUF = 640
NVB = BUF // 16
IMAX = 2147483647


def _pool_body(tok_ref, mask_ref, qe_ref):
    m = mask_ref[...].astype(jnp.float32)
    s = jnp.sum(tok_ref[...] * m[..., None], axis=1)
    qe_ref[...] = s / jnp.sum(m, axis=1)[:, None]


def _scores_body(qe_ref, corpus_ref, out_ref):
    i = pl.program_id(0)
    s = lax.dot_general(qe_ref[...], corpus_ref[...],
                        (((1,), (1,)), ((), ())),
                        preferred_element_type=jnp.float32)
    col = i * BLK + lax.broadcasted_iota(jnp.int32, s.shape, 1)
    out_ref[...] = jnp.where(col < N, s, jnp.float32(NEG))


def _fused_scores_body(tok_ref, mask_ref, corpus_ref, out_ref, qe_ref):
    i = pl.program_id(0)

    @pl.when(i == 0)
    def _():
        m = mask_ref[...].astype(jnp.float32)
        s = jnp.sum(tok_ref[...] * m[..., None], axis=1)
        qe_ref[...] = s / jnp.sum(m, axis=1)[:, None]

    s = lax.dot_general(qe_ref[...], corpus_ref[...],
                        (((1,), (1,)), ((), ())),
                        preferred_element_type=jnp.float32)
    col = i * BLK + lax.broadcasted_iota(jnp.int32, s.shape, 1)
    out_ref[...] = jnp.where(col < N, s, jnp.float32(NEG))


def _splat_f(x):
    return jnp.broadcast_to(x, (16,)).astype(jnp.float32)


def _splat(x, dtype):
    return jnp.broadcast_to(x, (16,)).astype(dtype)


def _topk_body(scores_hbm, out_val_hbm, out_idx_hbm,
               chunk_a, chunk_b, cval, cidx, csort, tval, tidx, oval, oidx,
               sem_a, sem_b):
    wid = lax.axis_index("s") * 2 + lax.axis_index("c")

    def rebuild(t_off):
        _, off = t_off
        def conv_body(j, _):
            v = cval[pl.ds(j * 16, 16)]
            b = plsc.bitcast(v, jnp.uint32)
            s = jnp.where(b >= jnp.uint32(0x80000000), ~b,
                          b | jnp.uint32(0x80000000))
            csort[pl.ds(j * 16, 16)] = s
            return 0
        lax.fori_loop(0, NVB, conv_body, 0)

        def bs_body(_, lohi):
            lo, hi = lohi
            mid = lo + ((hi - lo + jnp.uint32(1)) >> jnp.uint32(1))
            def cnt_body(j, acc):
                s = csort[pl.ds(j * 16, 16)]
                return acc + (s >= _splat(mid, jnp.uint32)).astype(jnp.int32)
            acc = lax.fori_loop(0, NVB, cnt_body, jnp.zeros((16,), jnp.int32))
            cnt = jnp.sum(acc)
            return lax.cond(cnt >= K,
                            lambda: (mid, hi),
                            lambda: (lo, mid - jnp.uint32(1)))
        lo, _ = lax.fori_loop(0, 32, bs_body,
                              (jnp.uint32(0), jnp.uint32(0xFF800000)))

        lov = _splat(lo, jnp.uint32)
        bits = jnp.where((lov & jnp.uint32(0x80000000)) != jnp.uint32(0),
                         lov & jnp.uint32(0x7FFFFFFF), ~lov)
        new_t = jnp.max(plsc.bitcast(bits, jnp.float32))
        nts = _splat_f(new_t)

        def fill_body(j, _):
            tval[pl.ds(j * 16, 16)] = _splat_f(jnp.float32(NEG))
            return 0
        lax.fori_loop(0, NVB, fill_body, 0)

        def comp_body(j, o):
            v = cval[pl.ds(j * 16, 16)]
            ix = cidx[pl.ds(j * 16, 16)]
            m = v >= nts
            plsc.store_compressed(tval.at[pl.ds(o, 16)], v, mask=m)
            plsc.store_compressed(tidx.at[pl.ds(o, 16)], ix, mask=m)
            return o + jnp.sum(m.astype(jnp.int32))
        new_off = lax.fori_loop(0, NVB, comp_body, jnp.int32(0))

        def copy_body(j, _):
            cval[pl.ds(j * 16, 16)] = tval[pl.ds(j * 16, 16)]
            cidx[pl.ds(j * 16, 16)] = tidx[pl.ds(j * 16, 16)]
            return 0
        lax.fori_loop(0, NVB, copy_body, 0)
        return new_t, new_off

    def query_body(qi, _):
        q = wid * 2 + qi

        def init_body(j, _):
            cval[pl.ds(j * 16, 16)] = _splat_f(jnp.float32(NEG))
            return 0
        lax.fori_loop(0, NVB, init_body, 0)

        def start(buf, sem, c):
            pltpu.async_copy(scores_hbm.at[q, pl.ds(c * CH, CH)], buf, sem)

        def wait(buf, sem, c):
            pltpu.make_async_copy(scores_hbm.at[q, pl.ds(c * CH, CH)],
                                  buf, sem).wait()

        def scan_chunk(chunk, c, t_off):
            def group_body(g, t_off):
                t, off = t_off
                base = g * GRP * 16
                acc = _splat_f(jnp.float32(NEG))
                for j in range(GRP):
                    acc = jnp.maximum(acc, chunk[pl.ds(base + j * 16, 16)])
                hit = jnp.any(acc >= _splat_f(t))

                def slow(off):
                    ts = _splat_f(t)
                    o = off
                    for j in range(GRP):
                        v = chunk[pl.ds(base + j * 16, 16)]
                        m = v >= ts
                        iv = (_splat(c * CH + base + j * 16, jnp.int32)
                              + lax.iota(jnp.int32, 16))
                        plsc.store_compressed(cval.at[pl.ds(o, 16)], v, mask=m)
                        plsc.store_compressed(cidx.at[pl.ds(o, 16)], iv, mask=m)
                        o = o + jnp.sum(m.astype(jnp.int32))
                    return o

                off = lax.cond(hit, slow, lambda o: o, off)
                return lax.cond(off > CAP, rebuild, lambda x: x, (t, off))

            return lax.fori_loop(0, NGRP, group_body, t_off)

        start(chunk_a, sem_a, jnp.int32(0))

        def pair_body(cp, t_off):
            c0 = 2 * cp
            c1 = c0 + 1
            c2 = jnp.minimum(c1 + 1, NCH - 1)
            wait(chunk_a, sem_a, c0)
            start(chunk_b, sem_b, c1)
            t_off = scan_chunk(chunk_a, c0, t_off)
            wait(chunk_b, sem_b, c1)
            start(chunk_a, sem_a, c2)
            t_off = scan_chunk(chunk_b, c1, t_off)
            return t_off

        t, off = lax.fori_loop(0, NCH // 2, pair_body,
                               (jnp.float32(NEG), jnp.int32(0)))
        wait(chunk_a, sem_a, jnp.int32(NCH - 1))

        t, off = rebuild((t, off))
        nv = (off + jnp.int32(15)) >> jnp.int32(4)

        def ext_body(s, _):
            def max_body(j, acc):
                return jnp.maximum(acc, cval[pl.ds(j * 16, 16)])
            m = jnp.max(lax.fori_loop(0, nv, max_body,
                                      _splat_f(jnp.float32(NEG))))
            ms = _splat_f(m)

            def idx_body(j, acc):
                v = cval[pl.ds(j * 16, 16)]
                ix = cidx[pl.ds(j * 16, 16)]
                return jnp.minimum(acc, jnp.where(v == ms, ix,
                                                  _splat(IMAX, jnp.int32)))
            bi = jnp.min(lax.fori_loop(0, nv, idx_body,
                                       _splat(IMAX, jnp.int32)))
            bs = _splat(bi, jnp.int32)

            def clr_body(j, _):
                v = cval[pl.ds(j * 16, 16)]
                ix = cidx[pl.ds(j * 16, 16)]
                cval[pl.ds(j * 16, 16)] = jnp.where(
                    (v == ms) & (ix == bs), _splat_f(jnp.float32(NEG)), v)
                return 0
            lax.fori_loop(0, nv, clr_body, 0)

            lane = lax.iota(jnp.int32, 16) == (s & jnp.int32(15))
            sv = _splat(s, jnp.int32)
            plsc.store_scatter(oval, [sv], ms, mask=lane)
            plsc.store_scatter(oidx, [sv], bs, mask=lane)
            return 0
        lax.fori_loop(0, K, ext_body, 0)

        pltpu.sync_copy(oval, out_val_hbm.at[q])
        pltpu.sync_copy(oidx, out_idx_hbm.at[q])
        return 0

    lax.fori_loop(0, 2, query_body, 0)


def kernel(query_token_emb, query_mask, corpus_embeddings, num_retrieve):
    grid = NPAD // BLK
    scores = pl.pallas_call(
        _fused_scores_body,
        grid=(grid,),
        in_specs=[
            pl.BlockSpec(query_token_emb.shape, lambda i: (0, 0, 0)),
            pl.BlockSpec(query_mask.shape, lambda i: (0, 0)),
            pl.BlockSpec((BLK, D), lambda i: (i, 0)),
        ],
        out_specs=pl.BlockSpec((Q, BLK), lambda i: (0, i)),
        out_shape=jax.ShapeDtypeStruct((Q, NPAD), jnp.float32),
        scratch_shapes=[pltpu.VMEM((Q, D), jnp.float32)],
    )(query_token_emb, query_mask, corpus_embeddings)

    mesh = plsc.VectorSubcoreMesh(core_axis_name="c", subcore_axis_name="s",
                                  num_cores=2, num_subcores=16)
    topk = functools.partial(
        pl.kernel,
        out_type=(jax.ShapeDtypeStruct((Q, K), jnp.float32),
                  jax.ShapeDtypeStruct((Q, K), jnp.int32)),
        mesh=mesh,
        compiler_params=pltpu.CompilerParams(needs_layout_passes=False),
        scratch_types=[
            pltpu.VMEM((CH,), jnp.float32),
            pltpu.VMEM((CH,), jnp.float32),
            pltpu.VMEM((BUF,), jnp.float32),
            pltpu.VMEM((BUF,), jnp.int32),
            pltpu.VMEM((BUF,), jnp.uint32),
            pltpu.VMEM((BUF,), jnp.float32),
            pltpu.VMEM((BUF,), jnp.int32),
            pltpu.VMEM((K,), jnp.float32),
            pltpu.VMEM((K,), jnp.int32),
            pltpu.SemaphoreType.DMA,
            pltpu.SemaphoreType.DMA,
        ],
    )(_topk_body)
    top_vals, top_idx = topk(scores)
    return top_vals, top_idx
